# baseline (speedup 1.0000x reference)
import functools

import jax
import jax.numpy as jnp
from jax import lax
from jax.experimental import pallas as pl
from jax.experimental.pallas import tpu as pltpu
from jax.experimental.pallas import tpu_sc as plsc


def _enorm_body(e_ref, out_ref):
    e = e_ref[...]
    nrm = jnp.sqrt(jnp.sum(e * e, axis=1, keepdims=True))
    out_ref[...] = (e / jnp.maximum(nrm, 1e-12)).astype(out_ref.dtype)


def _normalize_codebook(table):
    c, d = table.shape
    return pl.pallas_call(
        _enorm_body,
        out_shape=jax.ShapeDtypeStruct((c, d), jnp.bfloat16),
    )(table)


def _argmax_body(x_ref, e_n_ref, ind_ref, run_max_ref, run_idx_ref,
                 *, ck, nchunks):
    x = x_ref[...]
    nrm = jnp.sqrt(jnp.sum(x * x, axis=1, keepdims=True))
    xn = (x / jnp.maximum(nrm, 1e-12)).astype(e_n_ref.dtype)

    run_max_ref[...] = jnp.full_like(run_max_ref, -jnp.inf)
    run_idx_ref[...] = jnp.zeros_like(run_idx_ref)

    tm = x.shape[0]
    col = lax.broadcasted_iota(jnp.int32, (tm, ck), 1).astype(jnp.float32)

    def body(c, _):
        eblk = e_n_ref[pl.ds(c * ck, ck), :]
        s = lax.dot_general(xn, eblk, (((1,), (1,)), ((), ())),
                            preferred_element_type=jnp.float32)
        m = jnp.max(s, axis=1, keepdims=True)
        idx = jnp.min(jnp.where(s >= m, col, jnp.float32(1e9)),
                      axis=1, keepdims=True)
        upd = m > run_max_ref[...]
        run_idx_ref[...] = jnp.where(
            upd, idx.astype(jnp.int32) + c * ck, run_idx_ref[...])
        run_max_ref[...] = jnp.where(upd, m, run_max_ref[...])
        return 0

    lax.fori_loop(0, nchunks, body, 0, unroll=True)
    ind_ref[0] = run_idx_ref[...]


def _argmax_indices(x_flat, e_n):
    m, d = x_flat.shape
    c = e_n.shape[0]
    tm = 512
    ck = 1024
    grid = m // tm
    kern = functools.partial(_argmax_body, ck=ck, nchunks=c // ck)
    return pl.pallas_call(
        kern,
        grid=(grid,),
        in_specs=[
            pl.BlockSpec((tm, d), lambda i: (i, 0)),
            pl.BlockSpec((c, d), lambda i: (0, 0)),
        ],
        out_specs=pl.BlockSpec((1, tm, 1), lambda i: (i, 0, 0)),
        out_shape=jax.ShapeDtypeStruct((grid, tm, 1), jnp.int32),
        scratch_shapes=[
            pltpu.VMEM((tm, 1), jnp.float32),
            pltpu.VMEM((tm, 1), jnp.int32),
        ],
    )(x_flat, e_n)


def _make_sc_gather(v, d, m):
    info = plsc.get_sparse_core_info()
    nc, ns = info.num_cores, info.num_subcores
    nw = nc * ns
    rows_per_w = m // nw
    chunk = 128
    nchunk = rows_per_w // chunk
    mesh = plsc.VectorSubcoreMesh(core_axis_name="c", subcore_axis_name="s")

    @functools.partial(
        pl.kernel,
        mesh=mesh,
        out_type=jax.ShapeDtypeStruct((m, d), jnp.float32),
        scratch_types=[
            pltpu.VMEM((nchunk, chunk), jnp.int32),
            pltpu.VMEM((chunk, d), jnp.float32),
            pltpu.VMEM((chunk, d), jnp.float32),
            pltpu.SemaphoreType.DMA,
            pltpu.SemaphoreType.DMA,
        ],
    )
    def gather_k(table_hbm, idx_hbm, out_hbm, idx_v, rows0, rows1, sem0, sem1):
        wid = lax.axis_index("s") * nc + lax.axis_index("c")
        base = wid * rows_per_w
        pltpu.sync_copy(idx_hbm.at[pl.ds(wid * nchunk, nchunk)], idx_v)
        bufs = (rows0, rows1)
        sems = (sem0, sem1)
        cps = [pltpu.async_copy(table_hbm.at[idx_v.at[0]], bufs[0], sems[0])]
        for j in range(nchunk):
            if j + 1 < nchunk:
                cps.append(pltpu.async_copy(
                    table_hbm.at[idx_v.at[j + 1]], bufs[(j + 1) % 2],
                    sems[(j + 1) % 2]))
            cps[j].wait()
            pltpu.sync_copy(bufs[j % 2],
                            out_hbm.at[pl.ds(base + j * chunk, chunk)])

    return gather_k


def kernel(x, embed):
    b, n, d = x.shape
    h, c, _ = embed.shape
    m = b * n
    x_flat = x.reshape(m, d).astype(jnp.float32)
    table = embed.reshape(c, d).astype(jnp.float32)

    e_n = _normalize_codebook(table)
    ind = _argmax_indices(x_flat, e_n)
    ind_flat = ind.reshape(m)

    gather = _make_sc_gather(c, d, m)
    quantize = gather(table, ind.reshape(m // 128, 128))

    return (quantize.reshape(b, n, d), ind_flat.reshape(b, n))


# speedup vs baseline: 1.0562x; 1.0562x over previous
import functools

import jax
import jax.numpy as jnp
from jax import lax
from jax.experimental import pallas as pl
from jax.experimental.pallas import tpu as pltpu
from jax.experimental.pallas import tpu_sc as plsc


def _argmax_body(x_ref, e_ref, ind_ref, e_n_ref, run_max_ref, run_idx_ref,
                 *, ck, nchunks):
    @pl.when(pl.program_id(0) == 0)
    def _():
        e = e_ref[...]
        nrm = jnp.sqrt(jnp.sum(e * e, axis=1, keepdims=True))
        e_n_ref[...] = (e / jnp.maximum(nrm, 1e-12)).astype(e_n_ref.dtype)

    x = x_ref[...]
    nrm = jnp.sqrt(jnp.sum(x * x, axis=1, keepdims=True))
    xn = (x / jnp.maximum(nrm, 1e-12)).astype(e_n_ref.dtype)

    run_max_ref[...] = jnp.full_like(run_max_ref, -jnp.inf)
    run_idx_ref[...] = jnp.zeros_like(run_idx_ref)

    tm = x.shape[0]
    col = lax.broadcasted_iota(jnp.int32, (tm, ck), 1).astype(jnp.float32)

    def body(c, _):
        eblk = e_n_ref[pl.ds(c * ck, ck), :]
        s = lax.dot_general(xn, eblk, (((1,), (1,)), ((), ())),
                            preferred_element_type=jnp.float32)
        m = jnp.max(s, axis=1, keepdims=True)
        idx = jnp.min(jnp.where(s >= m, col, jnp.float32(1e9)),
                      axis=1, keepdims=True)
        upd = m > run_max_ref[...]
        run_idx_ref[...] = jnp.where(
            upd, idx.astype(jnp.int32) + c * ck, run_idx_ref[...])
        run_max_ref[...] = jnp.where(upd, m, run_max_ref[...])
        return 0

    lax.fori_loop(0, nchunks, body, 0, unroll=True)
    ind_ref[0] = run_idx_ref[...]


def _argmax_indices(x_flat, table):
    m, d = x_flat.shape
    c = table.shape[0]
    tm = 1024
    ck = 1024
    grid = m // tm
    kern = functools.partial(_argmax_body, ck=ck, nchunks=c // ck)
    return pl.pallas_call(
        kern,
        grid=(grid,),
        in_specs=[
            pl.BlockSpec((tm, d), lambda i: (i, 0)),
            pl.BlockSpec((c, d), lambda i: (0, 0)),
        ],
        out_specs=pl.BlockSpec((1, tm, 1), lambda i: (i, 0, 0)),
        out_shape=jax.ShapeDtypeStruct((grid, tm, 1), jnp.int32),
        scratch_shapes=[
            pltpu.VMEM((c, d), jnp.bfloat16),
            pltpu.VMEM((tm, 1), jnp.float32),
            pltpu.VMEM((tm, 1), jnp.int32),
        ],
    )(x_flat, table)


def _make_sc_gather(v, d, m):
    info = plsc.get_sparse_core_info()
    nc, ns = info.num_cores, info.num_subcores
    nw = nc * ns
    rows_per_w = m // nw
    chunk = 128
    nchunk = rows_per_w // chunk
    mesh = plsc.VectorSubcoreMesh(core_axis_name="c", subcore_axis_name="s")

    @functools.partial(
        pl.kernel,
        mesh=mesh,
        out_type=jax.ShapeDtypeStruct((m, d), jnp.float32),
        scratch_types=[
            pltpu.VMEM((nchunk, chunk), jnp.int32),
            pltpu.VMEM((chunk, d), jnp.float32),
            pltpu.VMEM((chunk, d), jnp.float32),
            pltpu.SemaphoreType.DMA,
            pltpu.SemaphoreType.DMA,
        ],
    )
    def gather_k(table_hbm, idx_hbm, out_hbm, idx_v, rows0, rows1, sem0, sem1):
        wid = lax.axis_index("s") * nc + lax.axis_index("c")
        base = wid * rows_per_w
        pltpu.sync_copy(idx_hbm.at[pl.ds(wid * nchunk, nchunk)], idx_v)
        bufs = (rows0, rows1)
        sems = (sem0, sem1)
        cps = [pltpu.async_copy(table_hbm.at[idx_v.at[0]], bufs[0], sems[0])]
        for j in range(nchunk):
            if j + 1 < nchunk:
                cps.append(pltpu.async_copy(
                    table_hbm.at[idx_v.at[j + 1]], bufs[(j + 1) % 2],
                    sems[(j + 1) % 2]))
            cps[j].wait()
            pltpu.sync_copy(bufs[j % 2],
                            out_hbm.at[pl.ds(base + j * chunk, chunk)])

    return gather_k


def kernel(x, embed):
    b, n, d = x.shape
    h, c, _ = embed.shape
    m = b * n
    x_flat = x.reshape(m, d).astype(jnp.float32)
    table = embed.reshape(c, d).astype(jnp.float32)

    ind = _argmax_indices(x_flat, table)
    ind_flat = ind.reshape(m)

    gather = _make_sc_gather(c, d, m)
    quantize = gather(table, ind.reshape(m // 128, 128))

    return (quantize.reshape(b, n, d), ind_flat.reshape(b, n))


# speedup vs baseline: 1.1068x; 1.0479x over previous
import functools

import jax
import jax.numpy as jnp
from jax import lax
from jax.experimental import pallas as pl
from jax.experimental.pallas import tpu as pltpu
from jax.experimental.pallas import tpu_sc as plsc


def _argmax_body(x_ref, e_ref, ind_ref, e_n_ref, run_max_ref, run_idx_ref,
                 *, ck, nchunks):
    @pl.when(pl.program_id(0) == 0)
    def _():
        e = e_ref[...]
        nrm = jnp.sqrt(jnp.sum(e * e, axis=1, keepdims=True))
        e_n_ref[...] = (e / jnp.maximum(nrm, 1e-12)).astype(e_n_ref.dtype)

    x = x_ref[...]
    nrm = jnp.sqrt(jnp.sum(x * x, axis=1, keepdims=True))
    xn = (x / jnp.maximum(nrm, 1e-12)).astype(e_n_ref.dtype)

    run_max_ref[...] = jnp.full_like(run_max_ref, -jnp.inf)
    run_idx_ref[...] = jnp.zeros_like(run_idx_ref)

    tm = x.shape[0]
    col = lax.broadcasted_iota(jnp.int32, (tm, ck), 1).astype(jnp.float32)

    def body(c, _):
        eblk = e_n_ref[pl.ds(c * ck, ck), :]
        s = lax.dot_general(xn, eblk, (((1,), (1,)), ((), ())),
                            preferred_element_type=jnp.float32)
        m = jnp.max(s, axis=1, keepdims=True)
        idx = jnp.min(jnp.where(s >= m, col, jnp.float32(1e9)),
                      axis=1, keepdims=True)
        upd = m > run_max_ref[...]
        run_idx_ref[...] = jnp.where(
            upd, idx.astype(jnp.int32) + c * ck, run_idx_ref[...])
        run_max_ref[...] = jnp.where(upd, m, run_max_ref[...])
        return 0

    lax.fori_loop(0, nchunks, body, 0, unroll=True)
    ind_ref[0] = run_idx_ref[...]


def _argmax_indices(x_flat, table):
    m, d = x_flat.shape
    c = table.shape[0]
    tm = 2048
    ck = 1024
    grid = m // tm
    kern = functools.partial(_argmax_body, ck=ck, nchunks=c // ck)
    return pl.pallas_call(
        kern,
        grid=(grid,),
        in_specs=[
            pl.BlockSpec((tm, d), lambda i: (i, 0)),
            pl.BlockSpec((c, d), lambda i: (0, 0)),
        ],
        out_specs=pl.BlockSpec((1, tm, 1), lambda i: (i, 0, 0)),
        out_shape=jax.ShapeDtypeStruct((grid, tm, 1), jnp.int32),
        scratch_shapes=[
            pltpu.VMEM((c, d), jnp.bfloat16),
            pltpu.VMEM((tm, 1), jnp.float32),
            pltpu.VMEM((tm, 1), jnp.int32),
        ],
    )(x_flat, table)


def _make_sc_gather(v, d, m):
    info = plsc.get_sparse_core_info()
    nc, ns = info.num_cores, info.num_subcores
    nw = nc * ns
    rows_per_w = m // nw
    chunk = 128
    nchunk = rows_per_w // chunk
    mesh = plsc.VectorSubcoreMesh(core_axis_name="c", subcore_axis_name="s")

    @functools.partial(
        pl.kernel,
        mesh=mesh,
        out_type=jax.ShapeDtypeStruct((m, d), jnp.float32),
        scratch_types=[
            pltpu.VMEM((nchunk, chunk), jnp.int32),
            pltpu.VMEM((chunk, d), jnp.float32),
            pltpu.VMEM((chunk, d), jnp.float32),
            pltpu.SemaphoreType.DMA,
            pltpu.SemaphoreType.DMA,
        ],
    )
    def gather_k(table_hbm, idx_hbm, out_hbm, idx_v, rows0, rows1, sem0, sem1):
        wid = lax.axis_index("s") * nc + lax.axis_index("c")
        base = wid * rows_per_w
        pltpu.sync_copy(idx_hbm.at[pl.ds(wid * nchunk, nchunk)], idx_v)
        bufs = (rows0, rows1)
        sems = (sem0, sem1)
        cps = [pltpu.async_copy(table_hbm.at[idx_v.at[0]], bufs[0], sems[0])]
        for j in range(nchunk):
            if j + 1 < nchunk:
                cps.append(pltpu.async_copy(
                    table_hbm.at[idx_v.at[j + 1]], bufs[(j + 1) % 2],
                    sems[(j + 1) % 2]))
            cps[j].wait()
            pltpu.sync_copy(bufs[j % 2],
                            out_hbm.at[pl.ds(base + j * chunk, chunk)])

    return gather_k


def kernel(x, embed):
    b, n, d = x.shape
    h, c, _ = embed.shape
    m = b * n
    x_flat = x.reshape(m, d).astype(jnp.float32)
    table = embed.reshape(c, d).astype(jnp.float32)

    ind = _argmax_indices(x_flat, table)
    ind_flat = ind.reshape(m)

    gather = _make_sc_gather(c, d, m)
    quantize = gather(table, ind.reshape(m // 128, 128))

    return (quantize.reshape(b, n, d), ind_flat.reshape(b, n))
